# baseline (speedup 1.0000x reference)
import functools

import jax
import jax.numpy as jnp
from jax import lax
from jax.experimental import pallas as pl
from jax.experimental.pallas import tpu as pltpu
from jax.experimental.pallas import tpu_sc as plsc

D_E = 64
L_MAX = 2048

_cached = None


def _build():
    global _cached
    if _cached is not None:
        return _cached

    info = plsc.get_sparse_core_info()
    NC, NS, L = info.num_cores, info.num_subcores, info.num_lanes
    NW = NC * NS
    BPW = L_MAX // NW

    mesh = plsc.VectorSubcoreMesh(core_axis_name="c", subcore_axis_name="s")

    @functools.partial(
        pl.kernel,
        mesh=mesh,
        out_type=jax.ShapeDtypeStruct((L_MAX, D_E), jnp.float32),
        scratch_types=[
            pltpu.VMEM((BPW,), jnp.int32),
            pltpu.VMEM((BPW, D_E), jnp.float32),
            pltpu.VMEM((BPW, D_E), jnp.float32),
            pltpu.SemaphoreType.DMA,
            pltpu.SemaphoreType.DMA,
        ],
        compiler_params=pltpu.CompilerParams(use_tc_tiling_on_sc=False),
    )
    def emb(x_hbm, word_hbm, pos_hbm, out_hbm, idx_v, rows_v, pos_v, sem_g, sem_p):
        wid = lax.axis_index("s") * NC + lax.axis_index("c")
        base = wid * BPW
        pltpu.sync_copy(x_hbm.at[pl.ds(base, BPW)], idx_v)
        gather = pltpu.make_async_copy(word_hbm.at[idx_v], rows_v, sem_g)
        gather.start()
        pos_cp = pltpu.make_async_copy(pos_hbm.at[pl.ds(base, BPW)], pos_v, sem_p)
        pos_cp.start()
        gather.wait()
        pos_cp.wait()
        for r in range(BPW):
            for c in range(D_E // L):
                sl = pl.ds(c * L, L)
                rows_v[r, sl] = rows_v[r, sl] + pos_v[r, sl]
        pltpu.sync_copy(rows_v, out_hbm.at[pl.ds(base, BPW)])

    _cached = emb
    return emb


def kernel(x, word_table, pos_table):
    emb = _build()
    return emb(x.astype(jnp.int32), word_table, pos_table)


# speedup vs baseline: 1.7112x; 1.7112x over previous
import functools

import jax
import jax.numpy as jnp
from jax import lax
from jax.experimental import pallas as pl
from jax.experimental.pallas import tpu as pltpu
from jax.experimental.pallas import tpu_sc as plsc

D_E = 64
L_MAX = 2048

_cached = None


def _build():
    global _cached
    if _cached is not None:
        return _cached

    info = plsc.get_sparse_core_info()
    NC, NS, L = info.num_cores, info.num_subcores, info.num_lanes
    NW = NC * NS
    BPW = L_MAX // NW

    mesh = plsc.VectorSubcoreMesh(core_axis_name="c", subcore_axis_name="s")

    @functools.partial(
        pl.kernel,
        mesh=mesh,
        out_type=jax.ShapeDtypeStruct((L_MAX, D_E), jnp.float32),
        scratch_types=[
            pltpu.VMEM((BPW,), jnp.int32),
            pltpu.VMEM((BPW, D_E), jnp.float32),
            pltpu.VMEM((BPW, D_E), jnp.float32),
            pltpu.SemaphoreType.DMA,
            pltpu.SemaphoreType.DMA,
        ],
    )
    def emb(x_hbm, word_hbm, pos_hbm, out_hbm, idx_v, rows_v, pos_v, sem_g, sem_p):
        wid = lax.axis_index("s") * NC + lax.axis_index("c")
        base = wid * BPW
        pltpu.sync_copy(x_hbm.at[pl.ds(base, BPW)], idx_v)
        pos_cp = pltpu.make_async_copy(pos_hbm.at[pl.ds(base, BPW)], pos_v, sem_p)
        pos_cp.start()
        copies = []
        for r0 in range(0, BPW, L):
            vec = idx_v[pl.ds(r0, L)]
            for j in range(L):
                cp = pltpu.make_async_copy(
                    word_hbm.at[pl.ds(vec[j], 1)], rows_v.at[pl.ds(r0 + j, 1)], sem_g
                )
                cp.start()
                copies.append(cp)
        for cp in copies:
            cp.wait()
        pos_cp.wait()
        for r in range(BPW):
            for c in range(D_E // L):
                sl = pl.ds(c * L, L)
                rows_v[r, sl] = rows_v[r, sl] + pos_v[r, sl]
        pltpu.sync_copy(rows_v, out_hbm.at[pl.ds(base, BPW)])

    _cached = emb
    return emb


def kernel(x, word_table, pos_table):
    emb = _build()
    return emb(x.astype(jnp.int32), word_table, pos_table)
